# baseline (speedup 1.0000x reference)
import functools

import jax
import jax.numpy as jnp
from jax import lax
from jax.experimental import pallas as pl
from jax.experimental.pallas import tpu as pltpu
from jax.experimental.pallas import tpu_sc as plsc

_NUM_TIMESTEPS = 1000
_BETA_START = 1e-4
_BETA_END = 0.02
_TAB_PAD = 1024
_ROWS = 32


def _sc_qsample(x, t, noise, tabs):
    B, C, H, W = x.shape
    n_chunks = C * (H // _ROWS)
    per_c = H // _ROWS
    mesh = plsc.VectorSubcoreMesh(core_axis_name="c", subcore_axis_name="s")

    @functools.partial(
        pl.kernel,
        out_type=jax.ShapeDtypeStruct((B, C, H, W), jnp.float32),
        mesh=mesh,
        compiler_params=pltpu.CompilerParams(needs_layout_passes=False),
        scratch_types=[
            pltpu.VMEM((2, _TAB_PAD), jnp.float32),
            pltpu.VMEM((B,), jnp.int32),
            pltpu.VMEM((2, _ROWS, W), jnp.float32),
            pltpu.VMEM((2, _ROWS, W), jnp.float32),
            pltpu.VMEM((2, _ROWS, W), jnp.float32),
            pltpu.SemaphoreType.DMA,
            pltpu.SemaphoreType.DMA,
            pltpu.SemaphoreType.DMA,
            pltpu.SemaphoreType.DMA,
            pltpu.SemaphoreType.DMA,
            pltpu.SemaphoreType.DMA,
        ],
    )
    def body(x_hbm, t_hbm, n_hbm, tabs_hbm, o_hbm, tabs_v, t_v, xb, nb, ob,
             xs0, xs1, ns0, ns1, os0, os1):
        cid = lax.axis_index("c")
        sid = lax.axis_index("s")
        w = sid * 2 + cid

        pltpu.sync_copy(tabs_hbm, tabs_v)
        pltpu.sync_copy(t_hbm, t_v)
        widx = jnp.full((16,), w, jnp.int32)
        tw = plsc.load_gather(t_v, [widx])
        zeros = jnp.zeros((16,), jnp.int32)
        ones = jnp.ones((16,), jnp.int32)
        av = plsc.load_gather(tabs_v, [zeros, tw])
        bv = plsc.load_gather(tabs_v, [ones, tw])

        xsems = (xs0, xs1)
        nsems = (ns0, ns1)
        osems = (os0, os1)

        def in_copies(j, p):
            c = j // per_c
            r0 = (j % per_c) * _ROWS
            cpx = pltpu.make_async_copy(
                x_hbm.at[w, c, pl.ds(r0, _ROWS), :], xb.at[p], xsems[p])
            cpn = pltpu.make_async_copy(
                n_hbm.at[w, c, pl.ds(r0, _ROWS), :], nb.at[p], nsems[p])
            return cpx, cpn

        def out_copy(j, p):
            c = j // per_c
            r0 = (j % per_c) * _ROWS
            return pltpu.make_async_copy(
                ob.at[p], o_hbm.at[w, c, pl.ds(r0, _ROWS), :], osems[p])

        cpx, cpn = in_copies(0, 0)
        cpx.start()
        cpn.start()

        def step(k, _):
            for p in range(2):
                j = 2 * k + p

                @pl.when(j + 1 < n_chunks)
                def _():
                    cpx, cpn = in_copies(j + 1, 1 - p)
                    cpx.start()
                    cpn.start()

                cpx, cpn = in_copies(j, p)
                cpx.wait()
                cpn.wait()

                @pl.when(j >= 2)
                def _():
                    out_copy(j - 2, p).wait()

                def row(r, _):
                    for l in range(W // 16):
                        sl = pl.ds(l * 16, 16)
                        ob[p, r, sl] = av * xb[p, r, sl] + bv * nb[p, r, sl]
                    return 0

                lax.fori_loop(0, _ROWS, row, 0)
                out_copy(j, p).start()
            return 0

        lax.fori_loop(0, n_chunks // 2, step, 0)
        out_copy(n_chunks - 2, 0).wait()
        out_copy(n_chunks - 1, 1).wait()

    return body(x, t, noise, tabs)


def kernel(x_start, t, noise):
    betas = jnp.linspace(_BETA_START, _BETA_END, _NUM_TIMESTEPS, dtype=jnp.float32)
    ac = jnp.cumprod(1.0 - betas, axis=0)
    pad = jnp.zeros((_TAB_PAD - _NUM_TIMESTEPS,), jnp.float32)
    tabs = jnp.stack(
        [
            jnp.concatenate([jnp.sqrt(ac), pad]),
            jnp.concatenate([jnp.sqrt(1.0 - ac), pad]),
        ]
    )
    return _sc_qsample(x_start, t, noise, tabs)


# speedup vs baseline: 1.2902x; 1.2902x over previous
import functools

import jax
import jax.numpy as jnp
from jax import lax
from jax.experimental import pallas as pl
from jax.experimental.pallas import tpu as pltpu
from jax.experimental.pallas import tpu_sc as plsc

_NUM_TIMESTEPS = 1000
_BETA_START = 1e-4
_BETA_END = 0.02
_TAB_PAD = 1024


def _sc_gather_coeffs(tabs, t):
    B = t.shape[0]
    mesh = plsc.VectorSubcoreMesh(core_axis_name="c", subcore_axis_name="s")

    @functools.partial(
        pl.kernel,
        out_type=jax.ShapeDtypeStruct((2 * B,), jnp.float32),
        mesh=mesh,
        compiler_params=pltpu.CompilerParams(needs_layout_passes=False),
        scratch_types=[
            pltpu.VMEM((2, _TAB_PAD), jnp.float32),
            pltpu.VMEM((B,), jnp.int32),
            pltpu.VMEM((2 * B,), jnp.float32),
            pltpu.SemaphoreType.DMA,
            pltpu.SemaphoreType.DMA,
        ],
    )
    def gather_kernel(tabs_hbm, t_hbm, c_out, tabs_v, t_v, c_v, sem1, sem2):
        cid = lax.axis_index("c")
        sid = lax.axis_index("s")

        @pl.when(jnp.logical_and(cid == 0, sid == 0))
        def _():
            cp1 = pltpu.make_async_copy(tabs_hbm, tabs_v, sem1)
            cp2 = pltpu.make_async_copy(t_hbm, t_v, sem2)
            cp1.start()
            cp2.start()
            cp1.wait()
            cp2.wait()
            for i in range(B // 16):
                idx = t_v[pl.ds(i * 16, 16)]
                c_v[pl.ds(i * 16, 16)] = plsc.load_gather(tabs_v, [jnp.zeros((16,), jnp.int32), idx])
                c_v[pl.ds(B + i * 16, 16)] = plsc.load_gather(tabs_v, [jnp.ones((16,), jnp.int32), idx])
            pltpu.sync_copy(c_v, c_out)

    return gather_kernel(tabs, t)


def _fma_body(c_ref, x_ref, n_ref, o_ref):
    i = pl.program_id(0)
    B = pl.num_programs(0)
    a = c_ref[i]
    b = c_ref[B + i]
    o_ref[...] = a * x_ref[...] + b * n_ref[...]


def kernel(x_start, t, noise):
    B, C, H, W = x_start.shape

    betas = jnp.linspace(_BETA_START, _BETA_END, _NUM_TIMESTEPS, dtype=jnp.float32)
    ac = jnp.cumprod(1.0 - betas, axis=0)
    pad = jnp.zeros((_TAB_PAD - _NUM_TIMESTEPS,), jnp.float32)
    tabs = jnp.stack(
        [
            jnp.concatenate([jnp.sqrt(ac), pad]),
            jnp.concatenate([jnp.sqrt(1.0 - ac), pad]),
        ]
    )

    coeffs = _sc_gather_coeffs(tabs, t)

    blk = (1, C, H, W)
    idx = lambda i: (i, 0, 0, 0)
    out = pl.pallas_call(
        _fma_body,
        grid=(B,),
        in_specs=[
            pl.BlockSpec(memory_space=pltpu.SMEM),
            pl.BlockSpec(blk, idx),
            pl.BlockSpec(blk, idx),
        ],
        out_specs=pl.BlockSpec(blk, idx),
        out_shape=jax.ShapeDtypeStruct((B, C, H, W), jnp.float32),
    )(coeffs, x_start, noise)
    return out


# speedup vs baseline: 1.3057x; 1.0120x over previous
import functools

import jax
import jax.numpy as jnp
from jax import lax
from jax.experimental import pallas as pl
from jax.experimental.pallas import tpu as pltpu
from jax.experimental.pallas import tpu_sc as plsc

_NUM_TIMESTEPS = 1000
_BETA_START = 1e-4
_BETA_END = 0.02
_TAB_PAD = 1024


def _sc_gather_coeffs(tabs, t):
    B = t.shape[0]
    mesh = plsc.VectorSubcoreMesh(core_axis_name="c", subcore_axis_name="s", num_cores=1)

    @functools.partial(
        pl.kernel,
        out_type=jax.ShapeDtypeStruct((2 * B,), jnp.float32),
        mesh=mesh,
        compiler_params=pltpu.CompilerParams(needs_layout_passes=False),
        scratch_types=[
            pltpu.VMEM((2, _TAB_PAD), jnp.float32),
            pltpu.VMEM((B,), jnp.int32),
            pltpu.VMEM((2 * B,), jnp.float32),
            pltpu.SemaphoreType.DMA,
            pltpu.SemaphoreType.DMA,
        ],
    )
    def gather_kernel(tabs_hbm, t_hbm, c_out, tabs_v, t_v, c_v, sem1, sem2):
        cid = lax.axis_index("c")
        sid = lax.axis_index("s")

        @pl.when(jnp.logical_and(cid == 0, sid == 0))
        def _():
            cp1 = pltpu.make_async_copy(tabs_hbm, tabs_v, sem1)
            cp2 = pltpu.make_async_copy(t_hbm, t_v, sem2)
            cp1.start()
            cp2.start()
            cp1.wait()
            cp2.wait()
            for i in range(B // 16):
                idx = t_v[pl.ds(i * 16, 16)]
                c_v[pl.ds(i * 16, 16)] = plsc.load_gather(tabs_v, [jnp.zeros((16,), jnp.int32), idx])
                c_v[pl.ds(B + i * 16, 16)] = plsc.load_gather(tabs_v, [jnp.ones((16,), jnp.int32), idx])
            pltpu.sync_copy(c_v, c_out)

    return gather_kernel(tabs, t)


def _fma_body(c_ref, x_ref, n_ref, o_ref):
    i = pl.program_id(0)
    B = pl.num_programs(0)
    a = c_ref[i]
    b = c_ref[B + i]
    o_ref[...] = a * x_ref[...] + b * n_ref[...]


def kernel(x_start, t, noise):
    B, C, H, W = x_start.shape

    betas = jnp.linspace(_BETA_START, _BETA_END, _NUM_TIMESTEPS, dtype=jnp.float32)
    ac = jnp.cumprod(1.0 - betas, axis=0)
    pad = jnp.zeros((_TAB_PAD - _NUM_TIMESTEPS,), jnp.float32)
    tabs = jnp.stack(
        [
            jnp.concatenate([jnp.sqrt(ac), pad]),
            jnp.concatenate([jnp.sqrt(1.0 - ac), pad]),
        ]
    )

    coeffs = _sc_gather_coeffs(tabs, t)

    blk = (1, C, H, W)
    idx = lambda i: (i, 0, 0, 0)
    out = pl.pallas_call(
        _fma_body,
        grid=(B,),
        in_specs=[
            pl.BlockSpec(memory_space=pltpu.SMEM),
            pl.BlockSpec(blk, idx),
            pl.BlockSpec(blk, idx),
        ],
        out_specs=pl.BlockSpec(blk, idx),
        out_shape=jax.ShapeDtypeStruct((B, C, H, W), jnp.float32),
    )(coeffs, x_start, noise)
    return out
